# baseline (speedup 1.0000x reference)
import functools

import jax
import jax.numpy as jnp
from jax import lax
from jax.experimental import pallas as pl
from jax.experimental.pallas import tpu as pltpu
from jax.experimental.pallas import tpu_sc as plsc

N = 10000
E = 320000
D = 128
EPS = 1e-9

NC = 2
NS = 16
NW = NC * NS
EPT = E // NW
K = 80
NCHUNK = EPT // K
RPTW = 640


NBUF = 4


def _spmm_tile(x_hbm, src_hbm, dst_hbm, vals_hbm, zeros_hbm, out_hbm,
               srcb, dstb, valb, rowb, acc_sh, *sems):
    isems = sems[0:NBUF]
    gsems = sems[NBUF:2 * NBUF]
    ssems = sems[2 * NBUF:3 * NBUF]
    c = lax.axis_index("c")
    s = lax.axis_index("s")
    wid = s * NC + c
    roff = jnp.minimum(s * RPTW, N - RPTW)

    pltpu.sync_copy(zeros_hbm, acc_sh.at[pl.ds(roff, RPTW)])
    plsc.subcore_barrier()

    def sdv_issue(j, b):
        ebase = wid * EPT + j * K
        pltpu.async_copy(src_hbm.at[pl.ds(ebase, K)], srcb.at[b], isems[b])
        pltpu.async_copy(dst_hbm.at[pl.ds(ebase, K)], dstb.at[b], isems[b])
        pltpu.async_copy(vals_hbm.at[pl.ds(ebase, K)], valb.at[b], isems[b])

    def sdv_wait(b):
        pltpu.make_async_copy(src_hbm.at[pl.ds(0, K)], srcb.at[b], isems[b]).wait()
        pltpu.make_async_copy(dst_hbm.at[pl.ds(0, K)], dstb.at[b], isems[b]).wait()
        pltpu.make_async_copy(vals_hbm.at[pl.ds(0, K)], valb.at[b], isems[b]).wait()

    def gather_issue(b):
        pltpu.async_copy(x_hbm.at[srcb.at[b]], rowb.at[b], gsems[b])

    def gather_wait(b):
        pltpu.make_async_copy(x_hbm.at[srcb.at[b]], rowb.at[b],
                              gsems[b]).wait()

    def scatter_issue(b):
        pltpu.async_copy(rowb.at[b], acc_sh.at[dstb.at[b]], ssems[b],
                         add=True)

    def scatter_wait(b):
        pltpu.make_async_copy(rowb.at[b], acc_sh.at[dstb.at[b]],
                              ssems[b]).wait()

    def scale(b):
        def scale16(k16, c2):
            vv = valb[b, pl.ds(k16 * 16, 16)]
            for l in range(16):
                v = vv[l]
                row = k16 * 16 + l
                for dd in range(D // 16):
                    sl = pl.ds(dd * 16, 16)
                    rowb[b, row, sl] = rowb[b, row, sl] * v
            return c2
        lax.fori_loop(0, K // 16, scale16, 0, unroll=False)

    sdv_issue(0, 0)
    sdv_issue(1, 1)
    sdv_wait(0)
    gather_issue(0)

    def section(j, p, has_next):
        b1 = (p + 1) % NBUF
        b2 = (p + 2) % NBUF
        bd = (p + NBUF - 2) % NBUF

        @pl.when(j >= 2)
        def _():
            scatter_wait(bd)

        if has_next:
            @pl.when(j + 2 < NCHUNK)
            def _():
                sdv_issue(j + 2, b2)

            @pl.when(j + 1 < NCHUNK)
            def _():
                sdv_wait(b1)
                gather_issue(b1)

        gather_wait(p)
        scale(p)
        scatter_issue(p)

    def body(jj, carry):
        for p in range(NBUF):
            section(jj * NBUF + p, p, True)
        return carry

    NMAIN = (NCHUNK // NBUF) * NBUF
    lax.fori_loop(0, NCHUNK // NBUF, body, 0, unroll=False)
    for j in range(NMAIN, NCHUNK):
        section(j, j % NBUF, j + 1 < NCHUNK)

    scatter_wait((NCHUNK - 2) % NBUF)
    scatter_wait((NCHUNK - 1) % NBUF)

    plsc.subcore_barrier()
    pltpu.sync_copy(acc_sh.at[pl.ds(roff, RPTW)],
                    out_hbm.at[c, pl.ds(roff, RPTW)])


_spmm_call = pl.kernel(
    _spmm_tile,
    out_type=jax.ShapeDtypeStruct((NC, N, D), jnp.float32),
    mesh=plsc.VectorSubcoreMesh(core_axis_name="c", subcore_axis_name="s"),
    scratch_types=[
        pltpu.VMEM((NBUF, K), jnp.int32),
        pltpu.VMEM((NBUF, K), jnp.int32),
        pltpu.VMEM((NBUF, K), jnp.float32),
        pltpu.VMEM((NBUF, K, D), jnp.float32),
        pltpu.VMEM_SHARED((N, D), jnp.float32),
    ] + [pltpu.SemaphoreType.DMA] * (3 * NBUF),
)


def _spmm(x, src, dst, vals, zeros):
    return _spmm_call(x, src, dst, vals, zeros)


BLK = 2000


def _f_nl(x, W, b, off, sca):
    vw = jnp.dot(x, W, preferred_element_type=jnp.float32) + b
    vw = jnp.maximum(vw, 0.0)
    mean = jnp.mean(vw, axis=1, keepdims=True)
    var = jnp.mean((vw - mean) ** 2, axis=1, keepdims=True)
    inv = lax.rsqrt(var + EPS)
    return (vw - mean) * inv * sca + off


def _h01_body(v_ref, p1_ref, w0_ref, w1_ref, b0_ref, b1_ref,
              of0_ref, of1_ref, sc0_ref, sc1_ref, s1_ref, o_ref):
    s1 = p1_ref[0] + p1_ref[1]
    s1_ref[...] = s1
    h0 = _f_nl(v_ref[...], w0_ref[...], b0_ref[...], of0_ref[...], sc0_ref[...])
    h1 = _f_nl(s1, w1_ref[...], b1_ref[...], of1_ref[...], sc1_ref[...])
    o_ref[...] = h0 + h1


def _h01(vecs, p1, W0, W1, b0, b1, off0, off1, sca0, sca1):
    row_spec = pl.BlockSpec((BLK, D), lambda i: (i, 0))
    w_spec = pl.BlockSpec((D, D), lambda i: (0, 0))
    vec_spec = pl.BlockSpec((1, D), lambda i: (0, 0))
    return pl.pallas_call(
        _h01_body,
        grid=(N // BLK,),
        in_specs=[row_spec,
                  pl.BlockSpec((NC, BLK, D), lambda i: (0, i, 0)),
                  w_spec, w_spec,
                  vec_spec, vec_spec, vec_spec, vec_spec,
                  vec_spec, vec_spec],
        out_specs=[row_spec, row_spec],
        out_shape=[jax.ShapeDtypeStruct((N, D), jnp.float32),
                   jax.ShapeDtypeStruct((N, D), jnp.float32)],
    )(vecs, p1, W0, W1, b0.reshape(1, D), b1.reshape(1, D),
      off0, off1, sca0, sca1)


def _final_body(h01_ref, p2_ref, w2_ref, b2_ref, of2_ref, sc2_ref, o_ref):
    s2 = p2_ref[0] + p2_ref[1]
    h2 = _f_nl(s2, w2_ref[...], b2_ref[...], of2_ref[...], sc2_ref[...])
    o_ref[...] = h01_ref[...] + h2


def _final(h01, p2, W2, b2, off2, sca2):
    row_spec = pl.BlockSpec((BLK, D), lambda i: (i, 0))
    w_spec = pl.BlockSpec((D, D), lambda i: (0, 0))
    vec_spec = pl.BlockSpec((1, D), lambda i: (0, 0))
    return pl.pallas_call(
        _final_body,
        grid=(N // BLK,),
        in_specs=[row_spec,
                  pl.BlockSpec((NC, BLK, D), lambda i: (0, i, 0)),
                  w_spec, vec_spec, vec_spec, vec_spec],
        out_specs=row_spec,
        out_shape=jax.ShapeDtypeStruct((N, D), jnp.float32),
    )(h01, p2, W2, b2.reshape(1, D), off2, sca2)


def kernel(vecs, edge_index, edge_vals, W0, W1, W2, b0, b1, b2,
           off0, off1, off2, sca0, sca1, sca2):
    zeros = jnp.zeros((RPTW, D), jnp.float32)
    dst = edge_index[0]
    src = edge_index[1]
    p1 = _spmm(vecs, src, dst, edge_vals, zeros)
    s1, h01 = _h01(vecs, p1, W0, W1, b0, b1, off0, off1, sca0, sca1)
    p2 = _spmm(s1, src, dst, edge_vals, zeros)
    return _final(h01, p2, W2, b2, off2, sca2)


# speedup vs baseline: 1.0059x; 1.0059x over previous
import functools

import jax
import jax.numpy as jnp
from jax import lax
from jax.experimental import pallas as pl
from jax.experimental.pallas import tpu as pltpu
from jax.experimental.pallas import tpu_sc as plsc

N = 10000
E = 320000
D = 128
EPS = 1e-9

NC = 2
NS = 16
NW = NC * NS
EPT = E // NW
K = 80
NCHUNK = EPT // K
RPTW = 640


NBUF = 4


def _spmm_tile(x_hbm, src_hbm, dst_hbm, vals_hbm, zeros_hbm, out_hbm,
               srcb, dstb, valb, rowb, acc_sh, *sems):
    isems = sems[0:NBUF]
    gsems = sems[NBUF:2 * NBUF]
    ssems = sems[2 * NBUF:3 * NBUF]
    c = lax.axis_index("c")
    s = lax.axis_index("s")
    wid = s * NC + c
    roff = jnp.minimum(s * RPTW, N - RPTW)

    pltpu.sync_copy(zeros_hbm, acc_sh.at[pl.ds(roff, RPTW)])
    plsc.subcore_barrier()

    def sdv_issue(j, b):
        ebase = wid * EPT + j * K
        pltpu.async_copy(src_hbm.at[pl.ds(ebase, K)], srcb.at[b], isems[b])
        pltpu.async_copy(dst_hbm.at[pl.ds(ebase, K)], dstb.at[b], isems[b])
        pltpu.async_copy(vals_hbm.at[pl.ds(ebase, K)], valb.at[b], isems[b])

    def sdv_wait(b):
        pltpu.make_async_copy(src_hbm.at[pl.ds(0, K)], srcb.at[b], isems[b]).wait()
        pltpu.make_async_copy(dst_hbm.at[pl.ds(0, K)], dstb.at[b], isems[b]).wait()
        pltpu.make_async_copy(vals_hbm.at[pl.ds(0, K)], valb.at[b], isems[b]).wait()

    def gather_issue(b):
        pltpu.async_copy(x_hbm.at[srcb.at[b]], rowb.at[b], gsems[b])

    def gather_wait(b):
        pltpu.make_async_copy(x_hbm.at[srcb.at[b]], rowb.at[b],
                              gsems[b]).wait()

    def scatter_issue(b):
        pltpu.async_copy(rowb.at[b], acc_sh.at[dstb.at[b]], ssems[b],
                         add=True)

    def scatter_wait(b):
        pltpu.make_async_copy(rowb.at[b], acc_sh.at[dstb.at[b]],
                              ssems[b]).wait()

    def scale(b):
        def scale16(k16, c2):
            vv = valb[b, pl.ds(k16 * 16, 16)]
            for l in range(16):
                v = vv[l]
                row = k16 * 16 + l
                for dd in range(D // 16):
                    sl = pl.ds(dd * 16, 16)
                    rowb[b, row, sl] = rowb[b, row, sl] * v
            return c2
        lax.fori_loop(0, K // 16, scale16, 0, unroll=False)

    sdv_issue(0, 0)
    sdv_issue(1, 1)
    sdv_wait(0)
    gather_issue(0)

    def section(j, p, has_next):
        b1 = (p + 1) % NBUF
        b2 = (p + 2) % NBUF
        bd = (p + NBUF - 2) % NBUF

        @pl.when(j >= 2)
        def _():
            scatter_wait(bd)

        if has_next:
            @pl.when(j + 2 < NCHUNK)
            def _():
                sdv_issue(j + 2, b2)

            @pl.when(j + 1 < NCHUNK)
            def _():
                sdv_wait(b1)
                gather_issue(b1)

        gather_wait(p)
        scale(p)
        scatter_issue(p)

    def body(jj, carry):
        for p in range(NBUF):
            section(jj * NBUF + p, p, True)
        return carry

    NMAIN = (NCHUNK // NBUF) * NBUF
    lax.fori_loop(0, NCHUNK // NBUF, body, 0, unroll=False)
    for j in range(NMAIN, NCHUNK):
        section(j, j % NBUF, j + 1 < NCHUNK)

    scatter_wait((NCHUNK - 2) % NBUF)
    scatter_wait((NCHUNK - 1) % NBUF)

    plsc.subcore_barrier()
    pltpu.sync_copy(acc_sh.at[pl.ds(roff, RPTW)],
                    out_hbm.at[c, pl.ds(roff, RPTW)])


_spmm_call = pl.kernel(
    _spmm_tile,
    out_type=jax.ShapeDtypeStruct((NC, N, D), jnp.float32),
    mesh=plsc.VectorSubcoreMesh(core_axis_name="c", subcore_axis_name="s"),
    scratch_types=[
        pltpu.VMEM((NBUF, K), jnp.int32),
        pltpu.VMEM((NBUF, K), jnp.int32),
        pltpu.VMEM((NBUF, K), jnp.float32),
        pltpu.VMEM((NBUF, K, D), jnp.float32),
        pltpu.VMEM_SHARED((N, D), jnp.float32),
    ] + [pltpu.SemaphoreType.DMA] * (3 * NBUF),
)


def _spmm(x, src, dst, vals, zeros):
    return _spmm_call(x, src, dst, vals, zeros)


BLK = 2000


def _add2_body(p_ref, o_ref):
    o_ref[...] = p_ref[0] + p_ref[1]


def _add2(p):
    return pl.pallas_call(
        _add2_body,
        grid=(N // BLK,),
        in_specs=[pl.BlockSpec((NC, BLK, D), lambda i: (0, i, 0))],
        out_specs=pl.BlockSpec((BLK, D), lambda i: (i, 0)),
        out_shape=jax.ShapeDtypeStruct((N, D), jnp.float32),
    )(p)


def _f_nl(x, W, b, off, sca):
    vw = jnp.dot(x, W, preferred_element_type=jnp.float32) + b
    vw = jnp.maximum(vw, 0.0)
    mean = jnp.mean(vw, axis=1, keepdims=True)
    var = jnp.mean((vw - mean) ** 2, axis=1, keepdims=True)
    inv = lax.rsqrt(var + EPS)
    return (vw - mean) * inv * sca + off


def _h01_body(v_ref, s1_ref, w0_ref, w1_ref, b0_ref, b1_ref,
              of0_ref, of1_ref, sc0_ref, sc1_ref, o_ref):
    h0 = _f_nl(v_ref[...], w0_ref[...], b0_ref[...], of0_ref[...], sc0_ref[...])
    h1 = _f_nl(s1_ref[...], w1_ref[...], b1_ref[...], of1_ref[...], sc1_ref[...])
    o_ref[...] = h0 + h1


def _h01(vecs, s1, W0, W1, b0, b1, off0, off1, sca0, sca1):
    row_spec = pl.BlockSpec((BLK, D), lambda i: (i, 0))
    w_spec = pl.BlockSpec((D, D), lambda i: (0, 0))
    vec_spec = pl.BlockSpec((1, D), lambda i: (0, 0))
    return pl.pallas_call(
        _h01_body,
        grid=(N // BLK,),
        in_specs=[row_spec, row_spec, w_spec, w_spec,
                  vec_spec, vec_spec, vec_spec, vec_spec,
                  vec_spec, vec_spec],
        out_specs=row_spec,
        out_shape=jax.ShapeDtypeStruct((N, D), jnp.float32),
    )(vecs, s1, W0, W1, b0.reshape(1, D), b1.reshape(1, D),
      off0, off1, sca0, sca1)


def _final_body(h01_ref, p2_ref, w2_ref, b2_ref, of2_ref, sc2_ref, o_ref):
    s2 = p2_ref[0] + p2_ref[1]
    h2 = _f_nl(s2, w2_ref[...], b2_ref[...], of2_ref[...], sc2_ref[...])
    o_ref[...] = h01_ref[...] + h2


def _final(h01, p2, W2, b2, off2, sca2):
    row_spec = pl.BlockSpec((BLK, D), lambda i: (i, 0))
    w_spec = pl.BlockSpec((D, D), lambda i: (0, 0))
    vec_spec = pl.BlockSpec((1, D), lambda i: (0, 0))
    return pl.pallas_call(
        _final_body,
        grid=(N // BLK,),
        in_specs=[row_spec,
                  pl.BlockSpec((NC, BLK, D), lambda i: (0, i, 0)),
                  w_spec, vec_spec, vec_spec, vec_spec],
        out_specs=row_spec,
        out_shape=jax.ShapeDtypeStruct((N, D), jnp.float32),
    )(h01, p2, W2, b2.reshape(1, D), off2, sca2)


def kernel(vecs, edge_index, edge_vals, W0, W1, W2, b0, b1, b2,
           off0, off1, off2, sca0, sca1, sca2):
    zeros = jnp.zeros((RPTW, D), jnp.float32)
    dst = edge_index[0]
    src = edge_index[1]
    p1 = _spmm(vecs, src, dst, edge_vals, zeros)
    s1 = _add2(p1)
    p2 = _spmm(s1, src, dst, edge_vals, zeros)
    h01 = _h01(vecs, s1, W0, W1, b0, b1, off0, off1, sca0, sca1)
    return _final(h01, p2, W2, b2, off2, sca2)
